# baseline (speedup 1.0000x reference)
import functools

import jax
import jax.numpy as jnp
from jax import lax
from jax.experimental import pallas as pl
from jax.experimental.pallas import tpu as pltpu
from jax.experimental.pallas import tpu_sc as plsc

N = 10000
E = 320000
G = 256
NPAD = 10240
ET = E + N
CHUNK = 256
NSUB = CHUNK // 128
CH = 42
EPT = CHUNK * CH
CHUNKB = 128
NSUBB = 1
CHB = EPT // CHUNKB
NW = 32
E_PAD = EPT * NW
NEG = -3.0e38

_mesh = plsc.VectorSubcoreMesh(core_axis_name="c", subcore_axis_name="s")
_sc_params = pltpu.CompilerParams(needs_layout_passes=False,
                                  use_tc_tiling_on_sc=False)


@functools.partial(
    pl.kernel,
    out_type=(jax.ShapeDtypeStruct((E_PAD,), jnp.float32),
              jax.ShapeDtypeStruct((2, NPAD), jnp.float32)),
    mesh=_mesh,
    scratch_types=[
        pltpu.VMEM((2, CHUNK), jnp.int32),
        pltpu.VMEM((2, CHUNK), jnp.int32),
        pltpu.VMEM((2, CHUNK, 64), jnp.float32),
        pltpu.VMEM((2, CHUNK, 64), jnp.float32),
        pltpu.VMEM((2, CHUNK), jnp.float32),
        pltpu.VMEM((NPAD,), jnp.float32),
        pltpu.VMEM((16, 640), jnp.float32),
        pltpu.VMEM((64,), jnp.float32),
        pltpu.VMEM((256,), jnp.float32),
        pltpu.VMEM((NPAD,), jnp.float32),
        pltpu.VMEM_SHARED((16, NPAD), jnp.float32),
        pltpu.SemaphoreType.DMA,
        pltpu.SemaphoreType.DMA,
        pltpu.SemaphoreType.DMA,
        pltpu.SemaphoreType.DMA,
        pltpu.SemaphoreType.DMA,
        pltpu.SemaphoreType.DMA,
    ],
    compiler_params=_sc_params,
)
def _edge_alpha(xl_hbm, xr_hbm, src_hbm, dst_hbm, att_hbm,
                alpha_hbm, amax_hbm,
                sidx, didx, xlg, xrg, alc, amax_p, mbuf, attv, stash, dupchk, ashr,
                semi0, semi1, semg0, semg1, sems0, sems1):
    cid = lax.axis_index("c")
    sid = lax.axis_index("s")
    wid = sid * 2 + cid
    base = wid * EPT
    semi = (semi0, semi1)
    semg = (semg0, semg1)
    sems = (sems0, sems1)

    def off_of(ch):
        return base + jnp.minimum(ch, CH - 1) * CHUNK

    def idx_start(ch, b):
        off = off_of(ch)
        pltpu.async_copy(src_hbm.at[pl.ds(off, CHUNK)], sidx.at[b], semi[b])
        pltpu.async_copy(dst_hbm.at[pl.ds(off, CHUNK)], didx.at[b], semi[b])

    def idx_wait(b):
        pltpu.make_async_copy(src_hbm.at[pl.ds(0, CHUNK)], sidx.at[b],
                              semi[b]).wait()
        pltpu.make_async_copy(dst_hbm.at[pl.ds(0, CHUNK)], didx.at[b],
                              semi[b]).wait()

    def gath_start(b):
        for s in range(NSUB):
            sl = pl.ds(s * 128, 128)
            pltpu.async_copy(xl_hbm.at[sidx.at[b, sl]], xlg.at[b, sl],
                             semg[b])
            pltpu.async_copy(xr_hbm.at[didx.at[b, sl]], xrg.at[b, sl],
                             semg[b])

    def gath_wait(b):
        for s in range(NSUB):
            sl = pl.ds(s * 128, 128)
            pltpu.make_async_copy(xl_hbm.at[sidx.at[b, sl]], xlg.at[b, sl],
                                  semg[b]).wait()
            pltpu.make_async_copy(xr_hbm.at[didx.at[b, sl]], xrg.at[b, sl],
                                  semg[b]).wait()

    def store_start(ch, b):
        pltpu.async_copy(alc.at[b], alpha_hbm.at[pl.ds(off_of(ch), CHUNK)],
                         sems[b])

    def store_wait(b):
        pltpu.make_async_copy(alc.at[b], alpha_hbm.at[pl.ds(0, CHUNK)],
                              sems[b]).wait()

    def _init(j, carry):
        amax_p[pl.ds(j * 16, 16)] = jnp.full((16,), NEG, jnp.float32)
        return carry
    lax.fori_loop(0, NPAD // 16, _init, 0)

    pltpu.sync_copy(att_hbm, attv)
    att4 = [attv[pl.ds(16 * i, 16)] for i in range(4)]
    lanes = lax.iota(jnp.int32, 16)
    i16 = lanes * 16

    idx_start(0, 0)
    idx_wait(0)
    gath_start(0)
    idx_start(1, 1)

    def compute(b):
        def _group(g, carry2):
            g16 = g * 16
            for l in range(16):
                e = g16 + l
                s = jnp.zeros((16,), jnp.float32)
                for cb in range(4):
                    z = (xlg[b, e, pl.ds(cb * 16, 16)]
                         + xrg[b, e, pl.ds(cb * 16, 16)])
                    s = s + att4[cb] * jnp.maximum(z, 0.2 * z)
                stash[pl.ds(l * 16, 16)] = s
            accs = [jnp.zeros((16,), jnp.float32) for _ in range(4)]
            for u in range(16):
                accs[u % 4] = accs[u % 4] + plsc.load_gather(stash, [i16 + u])
            acc = (accs[0] + accs[1]) + (accs[2] + accs[3])
            alc[b, pl.ds(g16, 16)] = acc

            dvec = didx[b, pl.ds(g16, 16)]
            flanes = lanes.astype(jnp.float32)
            plsc.store_scatter(dupchk, [dvec], flanes)
            back = plsc.load_gather(dupchk, [dvec])
            ndup = plsc.all_reduce_population_count(back != flanes)[0]

            @pl.when(ndup == 0)
            def _():
                cur = plsc.load_gather(amax_p, [dvec])
                plsc.store_scatter(amax_p, [dvec], jnp.maximum(cur, acc))

            @pl.when(ndup != 0)
            def _():
                for l in range(16):
                    cur = plsc.load_gather(amax_p, [dvec])
                    plsc.store_scatter(amax_p, [dvec],
                                       jnp.maximum(cur, acc),
                                       mask=lanes == l)
            return carry2
        lax.fori_loop(0, CHUNK // 16, _group, 0)

    def _pair(p, carry):
        for b in range(2):
            ch = 2 * p + b
            gath_wait(b)
            idx_wait(1 - b)
            gath_start(1 - b)

            @pl.when(ch >= 2)
            def _():
                store_wait(b)

            compute(b)
            store_start(ch, b)
            idx_start(ch + 2, b)
        return carry
    lax.fori_loop(0, CH // 2, _pair, 0)

    gath_wait(0)
    idx_wait(1)
    store_wait(0)
    store_wait(1)

    pltpu.sync_copy(amax_p, ashr.at[sid])
    plsc.subcore_barrier()
    pltpu.sync_copy(ashr.at[:, pl.ds(sid * 640, 640)], mbuf)

    def _red(j, carry):
        m = mbuf[0, pl.ds(j * 16, 16)]
        for i in range(1, 16):
            m = jnp.maximum(m, mbuf[i, pl.ds(j * 16, 16)])
        amax_p[pl.ds(j * 16, 16)] = m
        return carry
    lax.fori_loop(0, 640 // 16, _red, 0)
    pltpu.sync_copy(amax_p.at[pl.ds(0, 640)],
                    amax_hbm.at[cid, pl.ds(sid * 640, 640)])


@functools.partial(
    pl.kernel,
    out_type=jax.ShapeDtypeStruct((2, NPAD, 80), jnp.float32),
    mesh=_mesh,
    scratch_types=[
        pltpu.VMEM((2, CHUNKB), jnp.int32),
        pltpu.VMEM((2, CHUNKB), jnp.int32),
        pltpu.VMEM((2, CHUNKB), jnp.int32),
        pltpu.VMEM((2, CHUNKB, 64), jnp.float32),
        pltpu.VMEM((2, CHUNKB), jnp.float32),
        pltpu.VMEM((NPAD,), jnp.float32),
        pltpu.VMEM((NPAD,), jnp.float32),
        pltpu.VMEM((2, CHUNKB, 80), jnp.float32),
        pltpu.VMEM_SHARED((NPAD, 80), jnp.float32),
        pltpu.SemaphoreType.DMA,
        pltpu.SemaphoreType.DMA,
        pltpu.SemaphoreType.DMA,
        pltpu.SemaphoreType.DMA,
        pltpu.SemaphoreType.DMA,
        pltpu.SemaphoreType.DMA,
    ],
    compiler_params=_sc_params,
)
def _edge_scatter(xl_hbm, src_hbm, dst_hbm, alpha_hbm, amax_hbm, acc_hbm,
                  sidx, didx, didx_sc, xlg, alc, amax_m, tmp, stg, acc_shr,
                  semi0, semi1, semg0, semg1, semc0, semc1):
    cid = lax.axis_index("c")
    sid = lax.axis_index("s")
    wid = sid * 2 + cid
    base = wid * EPT
    lanes = lax.iota(jnp.int32, 16)
    zeros16 = jnp.zeros((16,), jnp.int32)
    semi = (semi0, semi1)
    semg = (semg0, semg1)
    semc = (semc0, semc1)

    def off_of(ch):
        return base + jnp.minimum(ch, CHB - 1) * CHUNKB

    def idx_start(ch, b):
        off = off_of(ch)
        pltpu.async_copy(src_hbm.at[pl.ds(off, CHUNKB)], sidx.at[b], semi[b])
        pltpu.async_copy(dst_hbm.at[pl.ds(off, CHUNKB)], didx.at[b], semi[b])
        pltpu.async_copy(alpha_hbm.at[pl.ds(off, CHUNKB)], alc.at[b], semi[b])

    def idx_wait(b):
        pltpu.make_async_copy(src_hbm.at[pl.ds(0, CHUNKB)], sidx.at[b],
                              semi[b]).wait()
        pltpu.make_async_copy(dst_hbm.at[pl.ds(0, CHUNKB)], didx.at[b],
                              semi[b]).wait()
        pltpu.make_async_copy(alpha_hbm.at[pl.ds(0, CHUNKB)], alc.at[b],
                              semi[b]).wait()

    def gath_start(b):
        for s in range(NSUBB):
            sl = pl.ds(s * 128, 128)
            pltpu.async_copy(xl_hbm.at[sidx.at[b, sl]], xlg.at[b, sl],
                             semg[b])

    def gath_wait(b):
        for s in range(NSUBB):
            sl = pl.ds(s * 128, 128)
            pltpu.make_async_copy(xl_hbm.at[sidx.at[b, sl]], xlg.at[b, sl],
                                  semg[b]).wait()

    def scat_start(b):
        pltpu.async_copy(stg.at[b], acc_shr.at[didx_sc.at[b]], semc[b],
                         add=True)

    def scat_wait(b):
        pltpu.make_async_copy(stg.at[b], acc_shr.at[didx_sc.at[b]],
                              semc[b]).wait()

    pltpu.sync_copy(amax_hbm.at[0], amax_m)
    pltpu.sync_copy(amax_hbm.at[1], tmp)

    def _mrg(j, carry):
        amax_m[pl.ds(j * 16, 16)] = jnp.maximum(amax_m[pl.ds(j * 16, 16)],
                                                tmp[pl.ds(j * 16, 16)])
        return carry
    lax.fori_loop(0, NPAD // 16, _mrg, 0)

    def _z(e, carry):
        for k in range(5):
            stg[0, e, pl.ds(k * 16, 16)] = jnp.zeros((16,), jnp.float32)
        return carry
    lax.fori_loop(0, CHUNKB, _z, 0)
    for j in range((640 + CHUNKB - 1) // CHUNKB):
        rows = min(CHUNKB, 640 - j * CHUNKB)
        pltpu.sync_copy(stg.at[0, pl.ds(0, rows)],
                        acc_shr.at[pl.ds(sid * 640 + j * CHUNKB, rows)])
    plsc.subcore_barrier()

    idx_start(0, 0)
    idx_wait(0)
    gath_start(0)
    idx_start(1, 1)

    def compute(b):
        for g in range(CHUNKB // 16):
            g16 = g * 16
            dvec = didx[b, pl.ds(g16, 16)]
            didx_sc[b, pl.ds(g16, 16)] = dvec
            mx = plsc.load_gather(amax_m, [dvec])
            alc[b, pl.ds(g16, 16)] = jnp.exp(alc[b, pl.ds(g16, 16)] - mx)

        def _row(r, carry2):
            for q in range(8):
                e = r * 8 + q
                ev = plsc.load_gather(alc.at[b], [zeros16 + e])
                for cb in range(4):
                    stg[b, e, pl.ds(cb * 16, 16)] = (
                        xlg[b, e, pl.ds(cb * 16, 16)] * ev)
                stg[b, e, pl.ds(64, 16)] = jnp.where(lanes == 0, ev, 0.0)
            return carry2
        lax.fori_loop(0, CHUNKB // 8, _row, 0)

    def _pair(p, carry):
        for b in range(2):
            ch = 2 * p + b
            gath_wait(b)
            idx_wait(1 - b)
            gath_start(1 - b)

            @pl.when(ch >= 2)
            def _():
                scat_wait(b)

            compute(b)
            scat_start(b)
            idx_start(ch + 2, b)
        return carry
    lax.fori_loop(0, CHB // 2, _pair, 0)

    gath_wait(0)
    idx_wait(1)
    scat_wait(0)
    scat_wait(1)

    plsc.subcore_barrier()
    pltpu.sync_copy(acc_shr.at[pl.ds(sid * 640, 640)],
                    acc_hbm.at[cid, pl.ds(sid * 640, 640)])


BLK = 1024
NBLK = NPAD // BLK


def _mm_body(x_ref, wl_ref, wr_ref, bl_ref, br_ref, xl_ref, xr_ref):
    xb = x_ref[...]
    xl_ref[...] = jnp.dot(xb, wl_ref[...],
                          preferred_element_type=jnp.float32) + bl_ref[...]
    xr_ref[...] = jnp.dot(xb, wr_ref[...],
                          preferred_element_type=jnp.float32) + br_ref[...]


def _mm(x_p, Wl, Wr, bl, br):
    din = x_p.shape[1]
    return pl.pallas_call(
        _mm_body,
        grid=(NBLK,),
        in_specs=[pl.BlockSpec((BLK, din), lambda i: (i, 0)),
                  pl.BlockSpec((din, 64), lambda i: (0, 0)),
                  pl.BlockSpec((din, 64), lambda i: (0, 0)),
                  pl.BlockSpec((1, 64), lambda i: (0, 0)),
                  pl.BlockSpec((1, 64), lambda i: (0, 0))],
        out_specs=[pl.BlockSpec((BLK, 64), lambda i: (i, 0)),
                   pl.BlockSpec((BLK, 64), lambda i: (i, 0))],
        out_shape=[jax.ShapeDtypeStruct((NPAD, 64), jnp.float32),
                   jax.ShapeDtypeStruct((NPAD, 64), jnp.float32)],
    )(x_p, Wl, Wr, bl.reshape(1, 64), br.reshape(1, 64))


def _fin_body(a0_ref, a1_ref, bias_ref, wl_ref, wr_ref, bl_ref, br_ref,
              xl_ref, xr_ref):
    a = a0_ref[...] + a1_ref[...]
    h = a[:, :64] / (a[:, 64:65] + 1e-16) + bias_ref[...]
    h = jnp.maximum(h, 0.0)
    xl_ref[...] = jnp.dot(h, wl_ref[...],
                          preferred_element_type=jnp.float32) + bl_ref[...]
    xr_ref[...] = jnp.dot(h, wr_ref[...],
                          preferred_element_type=jnp.float32) + br_ref[...]


def _fin(acc, bias, Wl, bl, Wr, br):
    return pl.pallas_call(
        _fin_body,
        grid=(NBLK,),
        in_specs=[pl.BlockSpec((BLK, 80), lambda i: (i, 0)),
                  pl.BlockSpec((BLK, 80), lambda i: (i, 0)),
                  pl.BlockSpec((1, 64), lambda i: (0, 0)),
                  pl.BlockSpec((64, 64), lambda i: (0, 0)),
                  pl.BlockSpec((64, 64), lambda i: (0, 0)),
                  pl.BlockSpec((1, 64), lambda i: (0, 0)),
                  pl.BlockSpec((1, 64), lambda i: (0, 0))],
        out_specs=[pl.BlockSpec((BLK, 64), lambda i: (i, 0)),
                   pl.BlockSpec((BLK, 64), lambda i: (i, 0))],
        out_shape=[jax.ShapeDtypeStruct((NPAD, 64), jnp.float32),
                   jax.ShapeDtypeStruct((NPAD, 64), jnp.float32)],
    )(acc[0], acc[1], bias.reshape(1, 64), Wl, Wr,
      bl.reshape(1, 64), br.reshape(1, 64))


def _pool_body(a0_ref, a1_ref, bias_ref, batch_ref, o_ref, sacc_ref):
    pid = pl.program_id(0)
    a = a0_ref[...] + a1_ref[...]
    h = a[:, :64] / (a[:, 64:65] + 1e-16) + bias_ref[...]
    h = jnp.where(lax.broadcasted_iota(jnp.int32, (BLK, 64), 1) == 63,
                  1.0, h)
    b = batch_ref[0]
    oh = jnp.where(b == lax.broadcasted_iota(jnp.int32, (G, BLK), 0),
                   1.0, 0.0)

    @pl.when(pid == 0)
    def _():
        sacc_ref[...] = jnp.zeros_like(sacc_ref)

    sacc_ref[...] += jnp.dot(oh, h, preferred_element_type=jnp.float32)

    @pl.when(pid == NBLK - 1)
    def _():
        s = sacc_ref[...]
        o_ref[...] = s / jnp.maximum(s[:, 63:64], 1.0)


def _pool(acc, bias, batch_p):
    return pl.pallas_call(
        _pool_body,
        grid=(NBLK,),
        in_specs=[pl.BlockSpec((BLK, 80), lambda i: (i, 0)),
                  pl.BlockSpec((BLK, 80), lambda i: (i, 0)),
                  pl.BlockSpec((1, 64), lambda i: (0, 0)),
                  pl.BlockSpec((1, 1, BLK), lambda i: (i, 0, 0))],
        out_specs=pl.BlockSpec((G, 64), lambda i: (0, 0)),
        out_shape=jax.ShapeDtypeStruct((G, 64), jnp.float32),
        scratch_shapes=[pltpu.VMEM((G, 64), jnp.float32)],
    )(acc[0], acc[1], bias.reshape(1, 64),
      batch_p.reshape(NBLK, 1, BLK))


def _pad_rows(a, rows):
    return jnp.pad(a, ((0, rows - a.shape[0]), (0, 0)))


def kernel(x, edge_index, batch, W_l0, b_l0, W_r0, b_r0, att0, bias0,
           W_l1, b_l1, W_r1, b_r1, att1, bias1,
           W_l2, b_l2, W_r2, b_r2, att2, bias2):
    loops = jnp.arange(N, dtype=jnp.int32)
    src = jnp.concatenate([edge_index[0].astype(jnp.int32), loops,
                           jnp.zeros((E_PAD - ET,), jnp.int32)])
    dst = jnp.concatenate([edge_index[1].astype(jnp.int32), loops,
                           jnp.full((E_PAD - ET,), N, jnp.int32)])
    batch_p = jnp.concatenate([batch.astype(jnp.int32),
                               jnp.full((NPAD - N,), 300, jnp.int32)])
    x_p = _pad_rows(x, NPAD)

    pad6 = lambda a: jnp.pad(a, ((0, 0), (0, 6)))
    Wl2, Wr2 = pad6(W_l2), pad6(W_r2)
    bl2 = jnp.pad(b_l2, (0, 6))
    br2 = jnp.pad(b_r2, (0, 6))
    att2p = jnp.pad(att2[0], (0, 6))
    bias2p = jnp.pad(bias2, (0, 6))

    xl, xr = _mm(x_p, W_l0, W_r0, b_l0, b_r0)
    alpha, amax_parts = _edge_alpha(xl, xr, src, dst, att0[0])
    acc = _edge_scatter(xl, src, dst, alpha, amax_parts)
    xl, xr = _fin(acc, bias0, W_l1, b_l1, W_r1, b_r1)
    alpha, amax_parts = _edge_alpha(xl, xr, src, dst, att1[0])
    acc = _edge_scatter(xl, src, dst, alpha, amax_parts)
    xl, xr = _fin(acc, bias1, Wl2, bl2, Wr2, br2)
    alpha, amax_parts = _edge_alpha(xl, xr, src, dst, att2p)
    acc = _edge_scatter(xl, src, dst, alpha, amax_parts)
    out = _pool(acc, bias2p, batch_p)
    return out[:, :58]


# speedup vs baseline: 1.2445x; 1.2445x over previous
import functools

import jax
import jax.numpy as jnp
from jax import lax
from jax.experimental import pallas as pl
from jax.experimental.pallas import tpu as pltpu
from jax.experimental.pallas import tpu_sc as plsc

N = 10000
E = 320000
G = 256
NPAD = 10240
ET = E + N
CHUNK = 256
NSUB = CHUNK // 128
CH = 42
EPT = CHUNK * CH
CHUNKB = 128
NSUBB = 1
CHB = EPT // CHUNKB
NW = 32
E_PAD = EPT * NW
NEG = -3.0e38

_mesh = plsc.VectorSubcoreMesh(core_axis_name="c", subcore_axis_name="s")
_sc_params = pltpu.CompilerParams(needs_layout_passes=False,
                                  use_tc_tiling_on_sc=False)


@functools.partial(
    pl.kernel,
    out_type=(jax.ShapeDtypeStruct((E_PAD,), jnp.float32),
              jax.ShapeDtypeStruct((2, NPAD), jnp.float32)),
    mesh=_mesh,
    scratch_types=[
        pltpu.VMEM((2, CHUNK), jnp.int32),
        pltpu.VMEM((2, CHUNK), jnp.int32),
        pltpu.VMEM((2, CHUNK, 64), jnp.bfloat16),
        pltpu.VMEM((2, CHUNK, 64), jnp.bfloat16),
        pltpu.VMEM((2, CHUNK), jnp.float32),
        pltpu.VMEM((NPAD,), jnp.float32),
        pltpu.VMEM((16, 640), jnp.float32),
        pltpu.VMEM((64,), jnp.float32),
        pltpu.VMEM((256,), jnp.float32),
        pltpu.VMEM((NPAD,), jnp.float32),
        pltpu.VMEM_SHARED((16, NPAD), jnp.float32),
        pltpu.SemaphoreType.DMA,
        pltpu.SemaphoreType.DMA,
        pltpu.SemaphoreType.DMA,
        pltpu.SemaphoreType.DMA,
        pltpu.SemaphoreType.DMA,
        pltpu.SemaphoreType.DMA,
    ],
    compiler_params=_sc_params,
)
def _edge_alpha(xl_hbm, xr_hbm, src_hbm, dst_hbm, att_hbm,
                alpha_hbm, amax_hbm,
                sidx, didx, xlg, xrg, alc, amax_p, mbuf, attv, stash, dupchk, ashr,
                semi0, semi1, semg0, semg1, sems0, sems1):
    cid = lax.axis_index("c")
    sid = lax.axis_index("s")
    wid = sid * 2 + cid
    base = wid * EPT
    semi = (semi0, semi1)
    semg = (semg0, semg1)
    sems = (sems0, sems1)

    def off_of(ch):
        return base + jnp.minimum(ch, CH - 1) * CHUNK

    def idx_start(ch, b):
        off = off_of(ch)
        pltpu.async_copy(src_hbm.at[pl.ds(off, CHUNK)], sidx.at[b], semi[b])
        pltpu.async_copy(dst_hbm.at[pl.ds(off, CHUNK)], didx.at[b], semi[b])

    def idx_wait(b):
        pltpu.make_async_copy(src_hbm.at[pl.ds(0, CHUNK)], sidx.at[b],
                              semi[b]).wait()
        pltpu.make_async_copy(dst_hbm.at[pl.ds(0, CHUNK)], didx.at[b],
                              semi[b]).wait()

    def gath_start(b):
        for s in range(NSUB):
            sl = pl.ds(s * 128, 128)
            pltpu.async_copy(xl_hbm.at[sidx.at[b, sl]], xlg.at[b, sl],
                             semg[b])
            pltpu.async_copy(xr_hbm.at[didx.at[b, sl]], xrg.at[b, sl],
                             semg[b])

    def gath_wait(b):
        for s in range(NSUB):
            sl = pl.ds(s * 128, 128)
            pltpu.make_async_copy(xl_hbm.at[sidx.at[b, sl]], xlg.at[b, sl],
                                  semg[b]).wait()
            pltpu.make_async_copy(xr_hbm.at[didx.at[b, sl]], xrg.at[b, sl],
                                  semg[b]).wait()

    def store_start(ch, b):
        pltpu.async_copy(alc.at[b], alpha_hbm.at[pl.ds(off_of(ch), CHUNK)],
                         sems[b])

    def store_wait(b):
        pltpu.make_async_copy(alc.at[b], alpha_hbm.at[pl.ds(0, CHUNK)],
                              sems[b]).wait()

    def _init(j, carry):
        amax_p[pl.ds(j * 16, 16)] = jnp.full((16,), NEG, jnp.float32)
        return carry
    lax.fori_loop(0, NPAD // 16, _init, 0)

    pltpu.sync_copy(att_hbm, attv)
    lanes = lax.iota(jnp.int32, 16)
    i16 = lanes * 16
    atte = [attv[pl.ds(32 * i, 16)] for i in range(2)]
    atto = [attv[pl.ds(32 * i + 16, 16)] for i in range(2)]

    idx_start(0, 0)
    idx_wait(0)
    gath_start(0)
    idx_start(1, 1)

    def compute(b):
        def _group(g, carry2):
            g16 = g * 16
            for l in range(16):
                e = g16 + l
                s = jnp.zeros((16,), jnp.float32)
                for cb in range(2):
                    z = (xlg[b, e, pl.ds(cb * 32, 32)]
                         + xrg[b, e, pl.ds(cb * 32, 32)])
                    lr = jnp.maximum(z, jnp.bfloat16(0.2) * z)
                    u0, u1 = plsc.unpack(
                        lr, format=plsc.PackFormat.INTERLEAVED,
                        preferred_element_type=jnp.float32)
                    s = s + atte[cb] * u0 + atto[cb] * u1
                stash[pl.ds(l * 16, 16)] = s
            accs = [jnp.zeros((16,), jnp.float32) for _ in range(4)]
            for u in range(16):
                accs[u % 4] = accs[u % 4] + plsc.load_gather(stash, [i16 + u])
            acc = (accs[0] + accs[1]) + (accs[2] + accs[3])
            alc[b, pl.ds(g16, 16)] = acc

            dvec = didx[b, pl.ds(g16, 16)]
            flanes = lanes.astype(jnp.float32)
            plsc.store_scatter(dupchk, [dvec], flanes)
            back = plsc.load_gather(dupchk, [dvec])
            ndup = plsc.all_reduce_population_count(back != flanes)[0]

            @pl.when(ndup == 0)
            def _():
                cur = plsc.load_gather(amax_p, [dvec])
                plsc.store_scatter(amax_p, [dvec], jnp.maximum(cur, acc))

            @pl.when(ndup != 0)
            def _():
                for l in range(16):
                    cur = plsc.load_gather(amax_p, [dvec])
                    plsc.store_scatter(amax_p, [dvec],
                                       jnp.maximum(cur, acc),
                                       mask=lanes == l)
            return carry2
        lax.fori_loop(0, CHUNK // 16, _group, 0)

    def _pair(p, carry):
        for b in range(2):
            ch = 2 * p + b
            gath_wait(b)
            idx_wait(1 - b)
            gath_start(1 - b)

            @pl.when(ch >= 2)
            def _():
                store_wait(b)

            compute(b)
            store_start(ch, b)
            idx_start(ch + 2, b)
        return carry
    lax.fori_loop(0, CH // 2, _pair, 0)

    gath_wait(0)
    idx_wait(1)
    store_wait(0)
    store_wait(1)

    pltpu.sync_copy(amax_p, ashr.at[sid])
    plsc.subcore_barrier()
    pltpu.sync_copy(ashr.at[:, pl.ds(sid * 640, 640)], mbuf)

    def _red(j, carry):
        m = mbuf[0, pl.ds(j * 16, 16)]
        for i in range(1, 16):
            m = jnp.maximum(m, mbuf[i, pl.ds(j * 16, 16)])
        amax_p[pl.ds(j * 16, 16)] = m
        return carry
    lax.fori_loop(0, 640 // 16, _red, 0)
    pltpu.sync_copy(amax_p.at[pl.ds(0, 640)],
                    amax_hbm.at[cid, pl.ds(sid * 640, 640)])


@functools.partial(
    pl.kernel,
    out_type=jax.ShapeDtypeStruct((2, NPAD, 80), jnp.float32),
    mesh=_mesh,
    scratch_types=[
        pltpu.VMEM((2, CHUNKB), jnp.int32),
        pltpu.VMEM((2, CHUNKB), jnp.int32),
        pltpu.VMEM((2, CHUNKB), jnp.int32),
        pltpu.VMEM((2, CHUNKB, 64), jnp.float32),
        pltpu.VMEM((2, CHUNKB), jnp.float32),
        pltpu.VMEM((NPAD,), jnp.float32),
        pltpu.VMEM((NPAD,), jnp.float32),
        pltpu.VMEM((2, CHUNKB, 80), jnp.float32),
        pltpu.VMEM_SHARED((NPAD, 80), jnp.float32),
        pltpu.SemaphoreType.DMA,
        pltpu.SemaphoreType.DMA,
        pltpu.SemaphoreType.DMA,
        pltpu.SemaphoreType.DMA,
        pltpu.SemaphoreType.DMA,
        pltpu.SemaphoreType.DMA,
    ],
    compiler_params=_sc_params,
)
def _edge_scatter(xl_hbm, src_hbm, dst_hbm, alpha_hbm, amax_hbm, acc_hbm,
                  sidx, didx, didx_sc, xlg, alc, amax_m, tmp, stg, acc_shr,
                  semi0, semi1, semg0, semg1, semc0, semc1):
    cid = lax.axis_index("c")
    sid = lax.axis_index("s")
    wid = sid * 2 + cid
    base = wid * EPT
    lanes = lax.iota(jnp.int32, 16)
    zeros16 = jnp.zeros((16,), jnp.int32)
    semi = (semi0, semi1)
    semg = (semg0, semg1)
    semc = (semc0, semc1)

    def off_of(ch):
        return base + jnp.minimum(ch, CHB - 1) * CHUNKB

    def idx_start(ch, b):
        off = off_of(ch)
        pltpu.async_copy(src_hbm.at[pl.ds(off, CHUNKB)], sidx.at[b], semi[b])
        pltpu.async_copy(dst_hbm.at[pl.ds(off, CHUNKB)], didx.at[b], semi[b])
        pltpu.async_copy(alpha_hbm.at[pl.ds(off, CHUNKB)], alc.at[b], semi[b])

    def idx_wait(b):
        pltpu.make_async_copy(src_hbm.at[pl.ds(0, CHUNKB)], sidx.at[b],
                              semi[b]).wait()
        pltpu.make_async_copy(dst_hbm.at[pl.ds(0, CHUNKB)], didx.at[b],
                              semi[b]).wait()
        pltpu.make_async_copy(alpha_hbm.at[pl.ds(0, CHUNKB)], alc.at[b],
                              semi[b]).wait()

    def gath_start(b):
        for s in range(NSUBB):
            sl = pl.ds(s * 128, 128)
            pltpu.async_copy(xl_hbm.at[sidx.at[b, sl]], xlg.at[b, sl],
                             semg[b])

    def gath_wait(b):
        for s in range(NSUBB):
            sl = pl.ds(s * 128, 128)
            pltpu.make_async_copy(xl_hbm.at[sidx.at[b, sl]], xlg.at[b, sl],
                                  semg[b]).wait()

    def scat_start(b):
        pltpu.async_copy(stg.at[b], acc_shr.at[didx_sc.at[b]], semc[b],
                         add=True)

    def scat_wait(b):
        pltpu.make_async_copy(stg.at[b], acc_shr.at[didx_sc.at[b]],
                              semc[b]).wait()

    pltpu.sync_copy(amax_hbm.at[0], amax_m)
    pltpu.sync_copy(amax_hbm.at[1], tmp)

    def _mrg(j, carry):
        amax_m[pl.ds(j * 16, 16)] = jnp.maximum(amax_m[pl.ds(j * 16, 16)],
                                                tmp[pl.ds(j * 16, 16)])
        return carry
    lax.fori_loop(0, NPAD // 16, _mrg, 0)

    def _z(e, carry):
        for k in range(5):
            stg[0, e, pl.ds(k * 16, 16)] = jnp.zeros((16,), jnp.float32)
        return carry
    lax.fori_loop(0, CHUNKB, _z, 0)
    for j in range((640 + CHUNKB - 1) // CHUNKB):
        rows = min(CHUNKB, 640 - j * CHUNKB)
        pltpu.sync_copy(stg.at[0, pl.ds(0, rows)],
                        acc_shr.at[pl.ds(sid * 640 + j * CHUNKB, rows)])
    plsc.subcore_barrier()

    idx_start(0, 0)
    idx_wait(0)
    gath_start(0)
    idx_start(1, 1)

    def compute(b):
        for g in range(CHUNKB // 16):
            g16 = g * 16
            dvec = didx[b, pl.ds(g16, 16)]
            didx_sc[b, pl.ds(g16, 16)] = dvec
            mx = plsc.load_gather(amax_m, [dvec])
            alc[b, pl.ds(g16, 16)] = jnp.exp(alc[b, pl.ds(g16, 16)] - mx)

        def _row(r, carry2):
            for q in range(8):
                e = r * 8 + q
                ev = plsc.load_gather(alc.at[b], [zeros16 + e])
                for cb in range(4):
                    stg[b, e, pl.ds(cb * 16, 16)] = (
                        xlg[b, e, pl.ds(cb * 16, 16)] * ev)
                stg[b, e, pl.ds(64, 16)] = jnp.where(lanes == 0, ev, 0.0)
            return carry2
        lax.fori_loop(0, CHUNKB // 8, _row, 0)

    def _pair(p, carry):
        for b in range(2):
            ch = 2 * p + b
            gath_wait(b)
            idx_wait(1 - b)
            gath_start(1 - b)

            @pl.when(ch >= 2)
            def _():
                scat_wait(b)

            compute(b)
            scat_start(b)
            idx_start(ch + 2, b)
        return carry
    lax.fori_loop(0, CHB // 2, _pair, 0)

    gath_wait(0)
    idx_wait(1)
    scat_wait(0)
    scat_wait(1)

    plsc.subcore_barrier()
    pltpu.sync_copy(acc_shr.at[pl.ds(sid * 640, 640)],
                    acc_hbm.at[cid, pl.ds(sid * 640, 640)])


BLK = 1024
NBLK = NPAD // BLK


def _mm_body(x_ref, wl_ref, wr_ref, bl_ref, br_ref, xl_ref, xr_ref,
             xlb_ref, xrb_ref):
    xb = x_ref[...]
    xl = jnp.dot(xb, wl_ref[...],
                 preferred_element_type=jnp.float32) + bl_ref[...]
    xr = jnp.dot(xb, wr_ref[...],
                 preferred_element_type=jnp.float32) + br_ref[...]
    xl_ref[...] = xl
    xr_ref[...] = xr
    xlb_ref[...] = xl.astype(jnp.bfloat16)
    xrb_ref[...] = xr.astype(jnp.bfloat16)


def _mm(x_p, Wl, Wr, bl, br):
    din = x_p.shape[1]
    return pl.pallas_call(
        _mm_body,
        grid=(NBLK,),
        in_specs=[pl.BlockSpec((BLK, din), lambda i: (i, 0)),
                  pl.BlockSpec((din, 64), lambda i: (0, 0)),
                  pl.BlockSpec((din, 64), lambda i: (0, 0)),
                  pl.BlockSpec((1, 64), lambda i: (0, 0)),
                  pl.BlockSpec((1, 64), lambda i: (0, 0))],
        out_specs=[pl.BlockSpec((BLK, 64), lambda i: (i, 0)),
                   pl.BlockSpec((BLK, 64), lambda i: (i, 0)),
                   pl.BlockSpec((BLK, 64), lambda i: (i, 0)),
                   pl.BlockSpec((BLK, 64), lambda i: (i, 0))],
        out_shape=[jax.ShapeDtypeStruct((NPAD, 64), jnp.float32),
                   jax.ShapeDtypeStruct((NPAD, 64), jnp.float32),
                   jax.ShapeDtypeStruct((NPAD, 64), jnp.bfloat16),
                   jax.ShapeDtypeStruct((NPAD, 64), jnp.bfloat16)],
    )(x_p, Wl, Wr, bl.reshape(1, 64), br.reshape(1, 64))


def _fin_body(a0_ref, a1_ref, bias_ref, wl_ref, wr_ref, bl_ref, br_ref,
              xl_ref, xr_ref, xlb_ref, xrb_ref):
    a = a0_ref[...] + a1_ref[...]
    h = a[:, :64] / (a[:, 64:65] + 1e-16) + bias_ref[...]
    h = jnp.maximum(h, 0.0)
    xl = jnp.dot(h, wl_ref[...],
                 preferred_element_type=jnp.float32) + bl_ref[...]
    xr = jnp.dot(h, wr_ref[...],
                 preferred_element_type=jnp.float32) + br_ref[...]
    xl_ref[...] = xl
    xr_ref[...] = xr
    xlb_ref[...] = xl.astype(jnp.bfloat16)
    xrb_ref[...] = xr.astype(jnp.bfloat16)


def _fin(acc, bias, Wl, bl, Wr, br):
    return pl.pallas_call(
        _fin_body,
        grid=(NBLK,),
        in_specs=[pl.BlockSpec((BLK, 80), lambda i: (i, 0)),
                  pl.BlockSpec((BLK, 80), lambda i: (i, 0)),
                  pl.BlockSpec((1, 64), lambda i: (0, 0)),
                  pl.BlockSpec((64, 64), lambda i: (0, 0)),
                  pl.BlockSpec((64, 64), lambda i: (0, 0)),
                  pl.BlockSpec((1, 64), lambda i: (0, 0)),
                  pl.BlockSpec((1, 64), lambda i: (0, 0))],
        out_specs=[pl.BlockSpec((BLK, 64), lambda i: (i, 0)),
                   pl.BlockSpec((BLK, 64), lambda i: (i, 0)),
                   pl.BlockSpec((BLK, 64), lambda i: (i, 0)),
                   pl.BlockSpec((BLK, 64), lambda i: (i, 0))],
        out_shape=[jax.ShapeDtypeStruct((NPAD, 64), jnp.float32),
                   jax.ShapeDtypeStruct((NPAD, 64), jnp.float32),
                   jax.ShapeDtypeStruct((NPAD, 64), jnp.bfloat16),
                   jax.ShapeDtypeStruct((NPAD, 64), jnp.bfloat16)],
    )(acc[0], acc[1], bias.reshape(1, 64), Wl, Wr,
      bl.reshape(1, 64), br.reshape(1, 64))


def _pool_body(a0_ref, a1_ref, bias_ref, batch_ref, o_ref, sacc_ref):
    pid = pl.program_id(0)
    a = a0_ref[...] + a1_ref[...]
    h = a[:, :64] / (a[:, 64:65] + 1e-16) + bias_ref[...]
    h = jnp.where(lax.broadcasted_iota(jnp.int32, (BLK, 64), 1) == 63,
                  1.0, h)
    b = batch_ref[0]
    oh = jnp.where(b == lax.broadcasted_iota(jnp.int32, (G, BLK), 0),
                   1.0, 0.0)

    @pl.when(pid == 0)
    def _():
        sacc_ref[...] = jnp.zeros_like(sacc_ref)

    sacc_ref[...] += jnp.dot(oh, h, preferred_element_type=jnp.float32)

    @pl.when(pid == NBLK - 1)
    def _():
        s = sacc_ref[...]
        o_ref[...] = s / jnp.maximum(s[:, 63:64], 1.0)


def _pool(acc, bias, batch_p):
    return pl.pallas_call(
        _pool_body,
        grid=(NBLK,),
        in_specs=[pl.BlockSpec((BLK, 80), lambda i: (i, 0)),
                  pl.BlockSpec((BLK, 80), lambda i: (i, 0)),
                  pl.BlockSpec((1, 64), lambda i: (0, 0)),
                  pl.BlockSpec((1, 1, BLK), lambda i: (i, 0, 0))],
        out_specs=pl.BlockSpec((G, 64), lambda i: (0, 0)),
        out_shape=jax.ShapeDtypeStruct((G, 64), jnp.float32),
        scratch_shapes=[pltpu.VMEM((G, 64), jnp.float32)],
    )(acc[0], acc[1], bias.reshape(1, 64),
      batch_p.reshape(NBLK, 1, BLK))


def _pad_rows(a, rows):
    return jnp.pad(a, ((0, rows - a.shape[0]), (0, 0)))


def kernel(x, edge_index, batch, W_l0, b_l0, W_r0, b_r0, att0, bias0,
           W_l1, b_l1, W_r1, b_r1, att1, bias1,
           W_l2, b_l2, W_r2, b_r2, att2, bias2):
    loops = jnp.arange(N, dtype=jnp.int32)
    src = jnp.concatenate([edge_index[0].astype(jnp.int32), loops,
                           jnp.zeros((E_PAD - ET,), jnp.int32)])
    dst = jnp.concatenate([edge_index[1].astype(jnp.int32), loops,
                           jnp.full((E_PAD - ET,), N, jnp.int32)])
    batch_p = jnp.concatenate([batch.astype(jnp.int32),
                               jnp.full((NPAD - N,), 300, jnp.int32)])
    x_p = _pad_rows(x, NPAD)

    pad6 = lambda a: jnp.pad(a, ((0, 0), (0, 6)))
    Wl2, Wr2 = pad6(W_l2), pad6(W_r2)
    bl2 = jnp.pad(b_l2, (0, 6))
    br2 = jnp.pad(b_r2, (0, 6))
    att2p = jnp.pad(att2[0], (0, 6))
    bias2p = jnp.pad(bias2, (0, 6))

    def att_perm(a):
        a2 = a.reshape(2, 16, 2)
        return jnp.concatenate([a2[:, :, 0], a2[:, :, 1]],
                               axis=1).reshape(64)

    xl, xr, xlb, xrb = _mm(x_p, W_l0, W_r0, b_l0, b_r0)
    alpha, amax_parts = _edge_alpha(xlb, xrb, src, dst, att_perm(att0[0]))
    acc = _edge_scatter(xl, src, dst, alpha, amax_parts)
    xl, xr, xlb, xrb = _fin(acc, bias0, W_l1, b_l1, W_r1, b_r1)
    alpha, amax_parts = _edge_alpha(xlb, xrb, src, dst, att_perm(att1[0]))
    acc = _edge_scatter(xl, src, dst, alpha, amax_parts)
    xl, xr, xlb, xrb = _fin(acc, bias1, Wl2, bl2, Wr2, br2)
    alpha, amax_parts = _edge_alpha(xlb, xrb, src, dst, att_perm(att2p))
    acc = _edge_scatter(xl, src, dst, alpha, amax_parts)
    out = _pool(acc, bias2p, batch_p)
    return out[:, :58]


# speedup vs baseline: 1.6254x; 1.3061x over previous
import functools

import jax
import jax.numpy as jnp
from jax import lax
from jax.experimental import pallas as pl
from jax.experimental.pallas import tpu as pltpu
from jax.experimental.pallas import tpu_sc as plsc

N = 10000
E = 320000
G = 256
NPAD = 10240
ET = E + N
CHUNK = 256
NSUB = CHUNK // 128
CH = 42
EPT = CHUNK * CH
CHUNKB = 128
NSUBB = 1
CHB = EPT // CHUNKB
NW = 32
E_PAD = EPT * NW
NEG = -3.0e38

_mesh = plsc.VectorSubcoreMesh(core_axis_name="c", subcore_axis_name="s")
_sc_params = pltpu.CompilerParams(needs_layout_passes=False,
                                  use_tc_tiling_on_sc=False)


@functools.partial(
    pl.kernel,
    out_type=(jax.ShapeDtypeStruct((E_PAD,), jnp.float32),
              jax.ShapeDtypeStruct((2, NPAD), jnp.float32)),
    mesh=_mesh,
    scratch_types=[
        pltpu.VMEM((2, CHUNK), jnp.int32),
        pltpu.VMEM((2, CHUNK), jnp.int32),
        pltpu.VMEM((2, CHUNK, 64), jnp.bfloat16),
        pltpu.VMEM((2, CHUNK, 64), jnp.bfloat16),
        pltpu.VMEM((2, CHUNK), jnp.float32),
        pltpu.VMEM((NPAD,), jnp.float32),
        pltpu.VMEM((16, 640), jnp.float32),
        pltpu.VMEM((64,), jnp.float32),
        pltpu.VMEM((256,), jnp.float32),
        pltpu.VMEM((NPAD,), jnp.float32),
        pltpu.VMEM_SHARED((16, NPAD), jnp.float32),
        pltpu.SemaphoreType.DMA,
        pltpu.SemaphoreType.DMA,
        pltpu.SemaphoreType.DMA,
        pltpu.SemaphoreType.DMA,
        pltpu.SemaphoreType.DMA,
        pltpu.SemaphoreType.DMA,
    ],
    compiler_params=_sc_params,
)
def _edge_alpha(xl_hbm, xr_hbm, src_hbm, dst_hbm, att_hbm,
                alpha_hbm, amax_hbm,
                sidx, didx, xlg, xrg, alc, amax_p, mbuf, attv, stash, dupchk, ashr,
                semi0, semi1, semg0, semg1, sems0, sems1):
    cid = lax.axis_index("c")
    sid = lax.axis_index("s")
    wid = sid * 2 + cid
    base = wid * EPT
    semi = (semi0, semi1)
    semg = (semg0, semg1)
    sems = (sems0, sems1)

    def off_of(ch):
        return base + jnp.minimum(ch, CH - 1) * CHUNK

    def idx_start(ch, b):
        off = off_of(ch)
        pltpu.async_copy(src_hbm.at[pl.ds(off, CHUNK)], sidx.at[b], semi[b])
        pltpu.async_copy(dst_hbm.at[pl.ds(off, CHUNK)], didx.at[b], semi[b])

    def idx_wait(b):
        pltpu.make_async_copy(src_hbm.at[pl.ds(0, CHUNK)], sidx.at[b],
                              semi[b]).wait()
        pltpu.make_async_copy(dst_hbm.at[pl.ds(0, CHUNK)], didx.at[b],
                              semi[b]).wait()

    def gath_start(b):
        for s in range(NSUB):
            sl = pl.ds(s * 128, 128)
            pltpu.async_copy(xl_hbm.at[sidx.at[b, sl]], xlg.at[b, sl],
                             semg[b])
            pltpu.async_copy(xr_hbm.at[didx.at[b, sl]], xrg.at[b, sl],
                             semg[b])

    def gath_wait(b):
        for s in range(NSUB):
            sl = pl.ds(s * 128, 128)
            pltpu.make_async_copy(xl_hbm.at[sidx.at[b, sl]], xlg.at[b, sl],
                                  semg[b]).wait()
            pltpu.make_async_copy(xr_hbm.at[didx.at[b, sl]], xrg.at[b, sl],
                                  semg[b]).wait()

    def store_start(ch, b):
        pltpu.async_copy(alc.at[b], alpha_hbm.at[pl.ds(off_of(ch), CHUNK)],
                         sems[b])

    def store_wait(b):
        pltpu.make_async_copy(alc.at[b], alpha_hbm.at[pl.ds(0, CHUNK)],
                              sems[b]).wait()

    def _init(j, carry):
        amax_p[pl.ds(j * 16, 16)] = jnp.full((16,), NEG, jnp.float32)
        return carry
    lax.fori_loop(0, NPAD // 16, _init, 0)

    pltpu.sync_copy(att_hbm, attv)
    lanes = lax.iota(jnp.int32, 16)
    i16 = lanes * 16
    atte = [attv[pl.ds(32 * i, 16)] for i in range(2)]
    atto = [attv[pl.ds(32 * i + 16, 16)] for i in range(2)]

    idx_start(0, 0)
    idx_wait(0)
    gath_start(0)
    idx_start(1, 1)

    def compute(b):
        def _group(g, carry2):
            g16 = g * 16
            for l in range(16):
                e = g16 + l
                s = jnp.zeros((16,), jnp.float32)
                for cb in range(2):
                    z = (xlg[b, e, pl.ds(cb * 32, 32)]
                         + xrg[b, e, pl.ds(cb * 32, 32)])
                    lr = jnp.maximum(z, jnp.bfloat16(0.2) * z)
                    u0, u1 = plsc.unpack(
                        lr, format=plsc.PackFormat.INTERLEAVED,
                        preferred_element_type=jnp.float32)
                    s = s + atte[cb] * u0 + atto[cb] * u1
                stash[pl.ds(l * 16, 16)] = s
            accs = [jnp.zeros((16,), jnp.float32) for _ in range(4)]
            for u in range(16):
                accs[u % 4] = accs[u % 4] + plsc.load_gather(stash, [i16 + u])
            acc = (accs[0] + accs[1]) + (accs[2] + accs[3])
            alc[b, pl.ds(g16, 16)] = acc

            dvec = didx[b, pl.ds(g16, 16)]
            flanes = lanes.astype(jnp.float32)
            plsc.store_scatter(dupchk, [dvec], flanes)
            back = plsc.load_gather(dupchk, [dvec])
            ndup = plsc.all_reduce_population_count(back != flanes)[0]

            @pl.when(ndup == 0)
            def _():
                cur = plsc.load_gather(amax_p, [dvec])
                plsc.store_scatter(amax_p, [dvec], jnp.maximum(cur, acc))

            @pl.when(ndup != 0)
            def _():
                for l in range(16):
                    cur = plsc.load_gather(amax_p, [dvec])
                    plsc.store_scatter(amax_p, [dvec],
                                       jnp.maximum(cur, acc),
                                       mask=lanes == l)
            return carry2
        lax.fori_loop(0, CHUNK // 16, _group, 0)

    def _pair(p, carry):
        for b in range(2):
            ch = 2 * p + b
            gath_wait(b)
            idx_wait(1 - b)
            gath_start(1 - b)

            @pl.when(ch >= 2)
            def _():
                store_wait(b)

            compute(b)
            store_start(ch, b)
            idx_start(ch + 2, b)
        return carry
    lax.fori_loop(0, CH // 2, _pair, 0)

    gath_wait(0)
    idx_wait(1)
    store_wait(0)
    store_wait(1)

    pltpu.sync_copy(amax_p, ashr.at[sid])
    plsc.subcore_barrier()
    pltpu.sync_copy(ashr.at[:, pl.ds(sid * 640, 640)], mbuf)

    def _red(j, carry):
        m = mbuf[0, pl.ds(j * 16, 16)]
        for i in range(1, 16):
            m = jnp.maximum(m, mbuf[i, pl.ds(j * 16, 16)])
        amax_p[pl.ds(j * 16, 16)] = m
        return carry
    lax.fori_loop(0, 640 // 16, _red, 0)
    pltpu.sync_copy(amax_p.at[pl.ds(0, 640)],
                    amax_hbm.at[cid, pl.ds(sid * 640, 640)])


@functools.partial(
    pl.kernel,
    out_type=jax.ShapeDtypeStruct((2, NPAD, 80), jnp.float32),
    mesh=_mesh,
    scratch_types=[
        pltpu.VMEM((2, CHUNKB), jnp.int32),
        pltpu.VMEM((2, CHUNKB), jnp.int32),
        pltpu.VMEM((2, CHUNKB), jnp.int32),
        pltpu.VMEM((2, CHUNKB, 64), jnp.bfloat16),
        pltpu.VMEM((2, CHUNKB), jnp.float32),
        pltpu.VMEM((NPAD,), jnp.float32),
        pltpu.VMEM((NPAD,), jnp.float32),
        pltpu.VMEM((2, CHUNKB, 80), jnp.float32),
        pltpu.VMEM_SHARED((NPAD, 80), jnp.float32),
        pltpu.SemaphoreType.DMA,
        pltpu.SemaphoreType.DMA,
        pltpu.SemaphoreType.DMA,
        pltpu.SemaphoreType.DMA,
        pltpu.SemaphoreType.DMA,
        pltpu.SemaphoreType.DMA,
    ],
    compiler_params=_sc_params,
)
def _edge_scatter(xl_hbm, src_hbm, dst_hbm, alpha_hbm, amax_hbm, acc_hbm,
                  sidx, didx, didx_sc, xlg, alc, amax_m, tmp, stg, acc_shr,
                  semi0, semi1, semg0, semg1, semc0, semc1):
    cid = lax.axis_index("c")
    sid = lax.axis_index("s")
    wid = sid * 2 + cid
    base = wid * EPT
    lanes = lax.iota(jnp.int32, 16)
    zeros16 = jnp.zeros((16,), jnp.int32)
    semi = (semi0, semi1)
    semg = (semg0, semg1)
    semc = (semc0, semc1)

    def off_of(ch):
        return base + jnp.minimum(ch, CHB - 1) * CHUNKB

    def idx_start(ch, b):
        off = off_of(ch)
        pltpu.async_copy(src_hbm.at[pl.ds(off, CHUNKB)], sidx.at[b], semi[b])
        pltpu.async_copy(dst_hbm.at[pl.ds(off, CHUNKB)], didx.at[b], semi[b])
        pltpu.async_copy(alpha_hbm.at[pl.ds(off, CHUNKB)], alc.at[b], semi[b])

    def idx_wait(b):
        pltpu.make_async_copy(src_hbm.at[pl.ds(0, CHUNKB)], sidx.at[b],
                              semi[b]).wait()
        pltpu.make_async_copy(dst_hbm.at[pl.ds(0, CHUNKB)], didx.at[b],
                              semi[b]).wait()
        pltpu.make_async_copy(alpha_hbm.at[pl.ds(0, CHUNKB)], alc.at[b],
                              semi[b]).wait()

    def gath_start(b):
        for s in range(NSUBB):
            sl = pl.ds(s * 128, 128)
            pltpu.async_copy(xl_hbm.at[sidx.at[b, sl]], xlg.at[b, sl],
                             semg[b])

    def gath_wait(b):
        for s in range(NSUBB):
            sl = pl.ds(s * 128, 128)
            pltpu.make_async_copy(xl_hbm.at[sidx.at[b, sl]], xlg.at[b, sl],
                                  semg[b]).wait()

    def scat_start(b):
        pltpu.async_copy(stg.at[b], acc_shr.at[didx_sc.at[b]], semc[b],
                         add=True)

    def scat_wait(b):
        pltpu.make_async_copy(stg.at[b], acc_shr.at[didx_sc.at[b]],
                              semc[b]).wait()

    pltpu.sync_copy(amax_hbm.at[0], amax_m)
    pltpu.sync_copy(amax_hbm.at[1], tmp)

    def _mrg(j, carry):
        amax_m[pl.ds(j * 16, 16)] = jnp.maximum(amax_m[pl.ds(j * 16, 16)],
                                                tmp[pl.ds(j * 16, 16)])
        return carry
    lax.fori_loop(0, NPAD // 16, _mrg, 0)

    def _z(e, carry):
        for k in range(5):
            stg[0, e, pl.ds(k * 16, 16)] = jnp.zeros((16,), jnp.float32)
        return carry
    lax.fori_loop(0, CHUNKB, _z, 0)
    for j in range((640 + CHUNKB - 1) // CHUNKB):
        rows = min(CHUNKB, 640 - j * CHUNKB)
        pltpu.sync_copy(stg.at[0, pl.ds(0, rows)],
                        acc_shr.at[pl.ds(sid * 640 + j * CHUNKB, rows)])
    plsc.subcore_barrier()

    idx_start(0, 0)
    idx_wait(0)
    gath_start(0)
    idx_start(1, 1)

    def compute(b):
        for g in range(CHUNKB // 16):
            g16 = g * 16
            dvec = didx[b, pl.ds(g16, 16)]
            didx_sc[b, pl.ds(g16, 16)] = dvec
            mx = plsc.load_gather(amax_m, [dvec])
            alc[b, pl.ds(g16, 16)] = jnp.exp(alc[b, pl.ds(g16, 16)] - mx)

        def _row(r, carry2):
            for q in range(8):
                e = r * 8 + q
                ev = plsc.load_gather(alc.at[b], [zeros16 + e])
                for cb in range(2):
                    v = xlg[b, e, pl.ds(cb * 32, 32)]
                    u0, u1 = plsc.unpack(
                        v, format=plsc.PackFormat.INTERLEAVED,
                        preferred_element_type=jnp.float32)
                    stg[b, e, pl.ds(cb * 32, 16)] = u0 * ev
                    stg[b, e, pl.ds(cb * 32 + 16, 16)] = u1 * ev
                stg[b, e, pl.ds(64, 16)] = jnp.where(lanes == 0, ev, 0.0)
            return carry2
        lax.fori_loop(0, CHUNKB // 8, _row, 0)

    def _pair(p, carry):
        for b in range(2):
            ch = 2 * p + b
            gath_wait(b)
            idx_wait(1 - b)
            gath_start(1 - b)

            @pl.when(ch >= 2)
            def _():
                scat_wait(b)

            compute(b)
            scat_start(b)
            idx_start(ch + 2, b)
        return carry
    lax.fori_loop(0, CHB // 2, _pair, 0)

    gath_wait(0)
    idx_wait(1)
    scat_wait(0)
    scat_wait(1)

    plsc.subcore_barrier()
    pltpu.sync_copy(acc_shr.at[pl.ds(sid * 640, 640)],
                    acc_hbm.at[cid, pl.ds(sid * 640, 640)])


BLK = 1024
NBLK = NPAD // BLK


def _mm_body(x_ref, wl_ref, wr_ref, bl_ref, br_ref, xl_ref, xr_ref,
             xlb_ref, xrb_ref):
    xb = x_ref[...]
    xl = jnp.dot(xb, wl_ref[...],
                 preferred_element_type=jnp.float32) + bl_ref[...]
    xr = jnp.dot(xb, wr_ref[...],
                 preferred_element_type=jnp.float32) + br_ref[...]
    xl_ref[...] = xl
    xr_ref[...] = xr
    xlb_ref[...] = xl.astype(jnp.bfloat16)
    xrb_ref[...] = xr.astype(jnp.bfloat16)


def _mm(x_p, Wl, Wr, bl, br):
    din = x_p.shape[1]
    return pl.pallas_call(
        _mm_body,
        grid=(NBLK,),
        in_specs=[pl.BlockSpec((BLK, din), lambda i: (i, 0)),
                  pl.BlockSpec((din, 64), lambda i: (0, 0)),
                  pl.BlockSpec((din, 64), lambda i: (0, 0)),
                  pl.BlockSpec((1, 64), lambda i: (0, 0)),
                  pl.BlockSpec((1, 64), lambda i: (0, 0))],
        out_specs=[pl.BlockSpec((BLK, 64), lambda i: (i, 0)),
                   pl.BlockSpec((BLK, 64), lambda i: (i, 0)),
                   pl.BlockSpec((BLK, 64), lambda i: (i, 0)),
                   pl.BlockSpec((BLK, 64), lambda i: (i, 0))],
        out_shape=[jax.ShapeDtypeStruct((NPAD, 64), jnp.float32),
                   jax.ShapeDtypeStruct((NPAD, 64), jnp.float32),
                   jax.ShapeDtypeStruct((NPAD, 64), jnp.bfloat16),
                   jax.ShapeDtypeStruct((NPAD, 64), jnp.bfloat16)],
    )(x_p, Wl, Wr, bl.reshape(1, 64), br.reshape(1, 64))


def _fin_body(a0_ref, a1_ref, bias_ref, wl_ref, wr_ref, bl_ref, br_ref,
              xl_ref, xr_ref, xlb_ref, xrb_ref):
    a = a0_ref[...] + a1_ref[...]
    h = a[:, :64] / (a[:, 64:65] + 1e-16) + bias_ref[...]
    h = jnp.maximum(h, 0.0)
    xl = jnp.dot(h, wl_ref[...],
                 preferred_element_type=jnp.float32) + bl_ref[...]
    xr = jnp.dot(h, wr_ref[...],
                 preferred_element_type=jnp.float32) + br_ref[...]
    xl_ref[...] = xl
    xr_ref[...] = xr
    xlb_ref[...] = xl.astype(jnp.bfloat16)
    xrb_ref[...] = xr.astype(jnp.bfloat16)


def _fin(acc, bias, Wl, bl, Wr, br):
    return pl.pallas_call(
        _fin_body,
        grid=(NBLK,),
        in_specs=[pl.BlockSpec((BLK, 80), lambda i: (i, 0)),
                  pl.BlockSpec((BLK, 80), lambda i: (i, 0)),
                  pl.BlockSpec((1, 64), lambda i: (0, 0)),
                  pl.BlockSpec((64, 64), lambda i: (0, 0)),
                  pl.BlockSpec((64, 64), lambda i: (0, 0)),
                  pl.BlockSpec((1, 64), lambda i: (0, 0)),
                  pl.BlockSpec((1, 64), lambda i: (0, 0))],
        out_specs=[pl.BlockSpec((BLK, 64), lambda i: (i, 0)),
                   pl.BlockSpec((BLK, 64), lambda i: (i, 0)),
                   pl.BlockSpec((BLK, 64), lambda i: (i, 0)),
                   pl.BlockSpec((BLK, 64), lambda i: (i, 0))],
        out_shape=[jax.ShapeDtypeStruct((NPAD, 64), jnp.float32),
                   jax.ShapeDtypeStruct((NPAD, 64), jnp.float32),
                   jax.ShapeDtypeStruct((NPAD, 64), jnp.bfloat16),
                   jax.ShapeDtypeStruct((NPAD, 64), jnp.bfloat16)],
    )(acc[0], acc[1], bias.reshape(1, 64), Wl, Wr,
      bl.reshape(1, 64), br.reshape(1, 64))


def _pool_body(a0_ref, a1_ref, bias_ref, batch_ref, o_ref, sacc_ref):
    pid = pl.program_id(0)
    a = a0_ref[...] + a1_ref[...]
    h = a[:, :64] / (a[:, 64:65] + 1e-16) + bias_ref[...]
    h = jnp.where(lax.broadcasted_iota(jnp.int32, (BLK, 64), 1) == 63,
                  1.0, h)
    b = batch_ref[0]
    oh = jnp.where(b == lax.broadcasted_iota(jnp.int32, (G, BLK), 0),
                   1.0, 0.0)

    @pl.when(pid == 0)
    def _():
        sacc_ref[...] = jnp.zeros_like(sacc_ref)

    sacc_ref[...] += jnp.dot(oh, h, preferred_element_type=jnp.float32)

    @pl.when(pid == NBLK - 1)
    def _():
        s = sacc_ref[...]
        o_ref[...] = s / jnp.maximum(s[:, 63:64], 1.0)


def _pool(acc, bias, batch_p):
    return pl.pallas_call(
        _pool_body,
        grid=(NBLK,),
        in_specs=[pl.BlockSpec((BLK, 80), lambda i: (i, 0)),
                  pl.BlockSpec((BLK, 80), lambda i: (i, 0)),
                  pl.BlockSpec((1, 64), lambda i: (0, 0)),
                  pl.BlockSpec((1, 1, BLK), lambda i: (i, 0, 0))],
        out_specs=pl.BlockSpec((G, 64), lambda i: (0, 0)),
        out_shape=jax.ShapeDtypeStruct((G, 64), jnp.float32),
        scratch_shapes=[pltpu.VMEM((G, 64), jnp.float32)],
    )(acc[0], acc[1], bias.reshape(1, 64),
      batch_p.reshape(NBLK, 1, BLK))


def _pad_rows(a, rows):
    return jnp.pad(a, ((0, rows - a.shape[0]), (0, 0)))


def kernel(x, edge_index, batch, W_l0, b_l0, W_r0, b_r0, att0, bias0,
           W_l1, b_l1, W_r1, b_r1, att1, bias1,
           W_l2, b_l2, W_r2, b_r2, att2, bias2):
    loops = jnp.arange(N, dtype=jnp.int32)
    src = jnp.concatenate([edge_index[0].astype(jnp.int32), loops,
                           jnp.zeros((E_PAD - ET,), jnp.int32)])
    dst = jnp.concatenate([edge_index[1].astype(jnp.int32), loops,
                           jnp.full((E_PAD - ET,), N, jnp.int32)])
    batch_p = jnp.concatenate([batch.astype(jnp.int32),
                               jnp.full((NPAD - N,), 300, jnp.int32)])
    x_p = _pad_rows(x, NPAD)

    pad6 = lambda a: jnp.pad(a, ((0, 0), (0, 6)))
    Wl2, Wr2 = pad6(W_l2), pad6(W_r2)
    bl2 = jnp.pad(b_l2, (0, 6))
    br2 = jnp.pad(b_r2, (0, 6))
    att2p = jnp.pad(att2[0], (0, 6))
    bias2p = jnp.pad(bias2, (0, 6))

    def att_perm(a):
        a2 = a.reshape(2, 16, 2)
        return jnp.concatenate([a2[:, :, 0], a2[:, :, 1]],
                               axis=1).reshape(64)

    perm = jnp.concatenate([jnp.arange(0, 32, 2), jnp.arange(1, 32, 2),
                            jnp.arange(32, 64, 2), jnp.arange(33, 64, 2)])
    inv = jnp.argsort(perm)

    xl, xr, xlb, xrb = _mm(x_p, W_l0, W_r0, b_l0, b_r0)
    alpha, amax_parts = _edge_alpha(xlb, xrb, src, dst, att_perm(att0[0]))
    acc = _edge_scatter(xlb, src, dst, alpha, amax_parts)
    xl, xr, xlb, xrb = _fin(acc, bias0[perm], W_l1[perm], b_l1,
                            W_r1[perm], b_r1)
    alpha, amax_parts = _edge_alpha(xlb, xrb, src, dst, att_perm(att1[0]))
    acc = _edge_scatter(xlb, src, dst, alpha, amax_parts)
    xl, xr, xlb, xrb = _fin(acc, bias1[perm], Wl2[perm], bl2,
                            Wr2[perm], br2)
    alpha, amax_parts = _edge_alpha(xlb, xrb, src, dst, att_perm(att2p))
    acc = _edge_scatter(xlb, src, dst, alpha, amax_parts)
    out = _pool(acc, bias2p[perm], batch_p)
    return out[:, inv][:, :58]


# speedup vs baseline: 1.6325x; 1.0043x over previous
import functools

import jax
import jax.numpy as jnp
from jax import lax
from jax.experimental import pallas as pl
from jax.experimental.pallas import tpu as pltpu
from jax.experimental.pallas import tpu_sc as plsc

N = 10000
E = 320000
G = 256
NPAD = 10240
ET = E + N
CHUNK = 384
NSUB = CHUNK // 128
CH = 28
EPT = CHUNK * CH
CHUNKB = 128
NSUBB = 1
CHB = EPT // CHUNKB
NW = 32
E_PAD = EPT * NW
NEG = -3.0e38

_mesh = plsc.VectorSubcoreMesh(core_axis_name="c", subcore_axis_name="s")
_sc_params = pltpu.CompilerParams(needs_layout_passes=False,
                                  use_tc_tiling_on_sc=False)


@functools.partial(
    pl.kernel,
    out_type=(jax.ShapeDtypeStruct((E_PAD,), jnp.float32),
              jax.ShapeDtypeStruct((2, NPAD), jnp.float32)),
    mesh=_mesh,
    scratch_types=[
        pltpu.VMEM((2, CHUNK), jnp.int32),
        pltpu.VMEM((2, CHUNK), jnp.int32),
        pltpu.VMEM((2, CHUNK, 64), jnp.bfloat16),
        pltpu.VMEM((2, CHUNK, 64), jnp.bfloat16),
        pltpu.VMEM((2, CHUNK), jnp.float32),
        pltpu.VMEM((NPAD,), jnp.float32),
        pltpu.VMEM((16, 640), jnp.float32),
        pltpu.VMEM((64,), jnp.float32),
        pltpu.VMEM((256,), jnp.float32),
        pltpu.VMEM((NPAD,), jnp.float32),
        pltpu.VMEM_SHARED((16, NPAD), jnp.float32),
        pltpu.SemaphoreType.DMA,
        pltpu.SemaphoreType.DMA,
        pltpu.SemaphoreType.DMA,
        pltpu.SemaphoreType.DMA,
        pltpu.SemaphoreType.DMA,
        pltpu.SemaphoreType.DMA,
    ],
    compiler_params=_sc_params,
)
def _edge_alpha(xl_hbm, xr_hbm, src_hbm, dst_hbm, att_hbm,
                alpha_hbm, amax_hbm,
                sidx, didx, xlg, xrg, alc, amax_p, mbuf, attv, stash, dupchk, ashr,
                semi0, semi1, semg0, semg1, sems0, sems1):
    cid = lax.axis_index("c")
    sid = lax.axis_index("s")
    wid = sid * 2 + cid
    base = wid * EPT
    semi = (semi0, semi1)
    semg = (semg0, semg1)
    sems = (sems0, sems1)

    def off_of(ch):
        return base + jnp.minimum(ch, CH - 1) * CHUNK

    def idx_start(ch, b):
        off = off_of(ch)
        pltpu.async_copy(src_hbm.at[pl.ds(off, CHUNK)], sidx.at[b], semi[b])
        pltpu.async_copy(dst_hbm.at[pl.ds(off, CHUNK)], didx.at[b], semi[b])

    def idx_wait(b):
        pltpu.make_async_copy(src_hbm.at[pl.ds(0, CHUNK)], sidx.at[b],
                              semi[b]).wait()
        pltpu.make_async_copy(dst_hbm.at[pl.ds(0, CHUNK)], didx.at[b],
                              semi[b]).wait()

    def gath_start(b):
        for s in range(NSUB):
            sl = pl.ds(s * 128, 128)
            pltpu.async_copy(xl_hbm.at[sidx.at[b, sl]], xlg.at[b, sl],
                             semg[b])
            pltpu.async_copy(xr_hbm.at[didx.at[b, sl]], xrg.at[b, sl],
                             semg[b])

    def gath_wait(b):
        for s in range(NSUB):
            sl = pl.ds(s * 128, 128)
            pltpu.make_async_copy(xl_hbm.at[sidx.at[b, sl]], xlg.at[b, sl],
                                  semg[b]).wait()
            pltpu.make_async_copy(xr_hbm.at[didx.at[b, sl]], xrg.at[b, sl],
                                  semg[b]).wait()

    def store_start(ch, b):
        pltpu.async_copy(alc.at[b], alpha_hbm.at[pl.ds(off_of(ch), CHUNK)],
                         sems[b])

    def store_wait(b):
        pltpu.make_async_copy(alc.at[b], alpha_hbm.at[pl.ds(0, CHUNK)],
                              sems[b]).wait()

    def _init(j, carry):
        amax_p[pl.ds(j * 16, 16)] = jnp.full((16,), NEG, jnp.float32)
        return carry
    lax.fori_loop(0, NPAD // 16, _init, 0)

    pltpu.sync_copy(att_hbm, attv)
    lanes = lax.iota(jnp.int32, 16)
    i16 = lanes * 16
    atte = [attv[pl.ds(32 * i, 16)] for i in range(2)]
    atto = [attv[pl.ds(32 * i + 16, 16)] for i in range(2)]

    idx_start(0, 0)
    idx_wait(0)
    gath_start(0)
    idx_start(1, 1)

    def compute(b):
        def _group(g, carry2):
            g16 = g * 16
            for l in range(16):
                e = g16 + l
                s = jnp.zeros((16,), jnp.float32)
                for cb in range(2):
                    z = (xlg[b, e, pl.ds(cb * 32, 32)]
                         + xrg[b, e, pl.ds(cb * 32, 32)])
                    lr = jnp.maximum(z, jnp.bfloat16(0.2) * z)
                    u0, u1 = plsc.unpack(
                        lr, format=plsc.PackFormat.INTERLEAVED,
                        preferred_element_type=jnp.float32)
                    s = s + atte[cb] * u0 + atto[cb] * u1
                stash[pl.ds(l * 16, 16)] = s
            accs = [jnp.zeros((16,), jnp.float32) for _ in range(4)]
            for u in range(16):
                accs[u % 4] = accs[u % 4] + plsc.load_gather(stash, [i16 + u])
            acc = (accs[0] + accs[1]) + (accs[2] + accs[3])
            alc[b, pl.ds(g16, 16)] = acc

            dvec = didx[b, pl.ds(g16, 16)]
            flanes = lanes.astype(jnp.float32)
            plsc.store_scatter(dupchk, [dvec], flanes)
            back = plsc.load_gather(dupchk, [dvec])
            ndup = plsc.all_reduce_population_count(back != flanes)[0]

            @pl.when(ndup == 0)
            def _():
                cur = plsc.load_gather(amax_p, [dvec])
                plsc.store_scatter(amax_p, [dvec], jnp.maximum(cur, acc))

            @pl.when(ndup != 0)
            def _():
                for l in range(16):
                    cur = plsc.load_gather(amax_p, [dvec])
                    plsc.store_scatter(amax_p, [dvec],
                                       jnp.maximum(cur, acc),
                                       mask=lanes == l)
            return carry2
        lax.fori_loop(0, CHUNK // 16, _group, 0)

    def _pair(p, carry):
        for b in range(2):
            ch = 2 * p + b
            gath_wait(b)
            idx_wait(1 - b)
            gath_start(1 - b)

            @pl.when(ch >= 2)
            def _():
                store_wait(b)

            compute(b)
            store_start(ch, b)
            idx_start(ch + 2, b)
        return carry
    lax.fori_loop(0, CH // 2, _pair, 0)

    gath_wait(0)
    idx_wait(1)
    store_wait(0)
    store_wait(1)

    pltpu.sync_copy(amax_p, ashr.at[sid])
    plsc.subcore_barrier()
    pltpu.sync_copy(ashr.at[:, pl.ds(sid * 640, 640)], mbuf)

    def _red(j, carry):
        m = mbuf[0, pl.ds(j * 16, 16)]
        for i in range(1, 16):
            m = jnp.maximum(m, mbuf[i, pl.ds(j * 16, 16)])
        amax_p[pl.ds(j * 16, 16)] = m
        return carry
    lax.fori_loop(0, 640 // 16, _red, 0)
    pltpu.sync_copy(amax_p.at[pl.ds(0, 640)],
                    amax_hbm.at[cid, pl.ds(sid * 640, 640)])


@functools.partial(
    pl.kernel,
    out_type=jax.ShapeDtypeStruct((2, NPAD, 80), jnp.float32),
    mesh=_mesh,
    scratch_types=[
        pltpu.VMEM((2, CHUNKB), jnp.int32),
        pltpu.VMEM((2, CHUNKB), jnp.int32),
        pltpu.VMEM((2, CHUNKB), jnp.int32),
        pltpu.VMEM((2, CHUNKB, 64), jnp.bfloat16),
        pltpu.VMEM((2, CHUNKB), jnp.float32),
        pltpu.VMEM((NPAD,), jnp.float32),
        pltpu.VMEM((NPAD,), jnp.float32),
        pltpu.VMEM((2, CHUNKB, 80), jnp.float32),
        pltpu.VMEM_SHARED((NPAD, 80), jnp.float32),
        pltpu.SemaphoreType.DMA,
        pltpu.SemaphoreType.DMA,
        pltpu.SemaphoreType.DMA,
        pltpu.SemaphoreType.DMA,
        pltpu.SemaphoreType.DMA,
        pltpu.SemaphoreType.DMA,
    ],
    compiler_params=_sc_params,
)
def _edge_scatter(xl_hbm, src_hbm, dst_hbm, alpha_hbm, amax_hbm, acc_hbm,
                  sidx, didx, didx_sc, xlg, alc, amax_m, tmp, stg, acc_shr,
                  semi0, semi1, semg0, semg1, semc0, semc1):
    cid = lax.axis_index("c")
    sid = lax.axis_index("s")
    wid = sid * 2 + cid
    base = wid * EPT
    lanes = lax.iota(jnp.int32, 16)
    zeros16 = jnp.zeros((16,), jnp.int32)
    semi = (semi0, semi1)
    semg = (semg0, semg1)
    semc = (semc0, semc1)

    def off_of(ch):
        return base + jnp.minimum(ch, CHB - 1) * CHUNKB

    def idx_start(ch, b):
        off = off_of(ch)
        pltpu.async_copy(src_hbm.at[pl.ds(off, CHUNKB)], sidx.at[b], semi[b])
        pltpu.async_copy(dst_hbm.at[pl.ds(off, CHUNKB)], didx.at[b], semi[b])
        pltpu.async_copy(alpha_hbm.at[pl.ds(off, CHUNKB)], alc.at[b], semi[b])

    def idx_wait(b):
        pltpu.make_async_copy(src_hbm.at[pl.ds(0, CHUNKB)], sidx.at[b],
                              semi[b]).wait()
        pltpu.make_async_copy(dst_hbm.at[pl.ds(0, CHUNKB)], didx.at[b],
                              semi[b]).wait()
        pltpu.make_async_copy(alpha_hbm.at[pl.ds(0, CHUNKB)], alc.at[b],
                              semi[b]).wait()

    def gath_start(b):
        for s in range(NSUBB):
            sl = pl.ds(s * 128, 128)
            pltpu.async_copy(xl_hbm.at[sidx.at[b, sl]], xlg.at[b, sl],
                             semg[b])

    def gath_wait(b):
        for s in range(NSUBB):
            sl = pl.ds(s * 128, 128)
            pltpu.make_async_copy(xl_hbm.at[sidx.at[b, sl]], xlg.at[b, sl],
                                  semg[b]).wait()

    def scat_start(b):
        pltpu.async_copy(stg.at[b], acc_shr.at[didx_sc.at[b]], semc[b],
                         add=True)

    def scat_wait(b):
        pltpu.make_async_copy(stg.at[b], acc_shr.at[didx_sc.at[b]],
                              semc[b]).wait()

    pltpu.sync_copy(amax_hbm.at[0], amax_m)
    pltpu.sync_copy(amax_hbm.at[1], tmp)

    def _mrg(j, carry):
        amax_m[pl.ds(j * 16, 16)] = jnp.maximum(amax_m[pl.ds(j * 16, 16)],
                                                tmp[pl.ds(j * 16, 16)])
        return carry
    lax.fori_loop(0, NPAD // 16, _mrg, 0)

    def _z(e, carry):
        for k in range(5):
            stg[0, e, pl.ds(k * 16, 16)] = jnp.zeros((16,), jnp.float32)
        return carry
    lax.fori_loop(0, CHUNKB, _z, 0)
    for j in range((640 + CHUNKB - 1) // CHUNKB):
        rows = min(CHUNKB, 640 - j * CHUNKB)
        pltpu.sync_copy(stg.at[0, pl.ds(0, rows)],
                        acc_shr.at[pl.ds(sid * 640 + j * CHUNKB, rows)])
    plsc.subcore_barrier()

    idx_start(0, 0)
    idx_wait(0)
    gath_start(0)
    idx_start(1, 1)

    def compute(b):
        for g in range(CHUNKB // 16):
            g16 = g * 16
            dvec = didx[b, pl.ds(g16, 16)]
            didx_sc[b, pl.ds(g16, 16)] = dvec
            mx = plsc.load_gather(amax_m, [dvec])
            alc[b, pl.ds(g16, 16)] = jnp.exp(alc[b, pl.ds(g16, 16)] - mx)

        def _row(r, carry2):
            for q in range(8):
                e = r * 8 + q
                ev = plsc.load_gather(alc.at[b], [zeros16 + e])
                for cb in range(2):
                    v = xlg[b, e, pl.ds(cb * 32, 32)]
                    u0, u1 = plsc.unpack(
                        v, format=plsc.PackFormat.INTERLEAVED,
                        preferred_element_type=jnp.float32)
                    stg[b, e, pl.ds(cb * 32, 16)] = u0 * ev
                    stg[b, e, pl.ds(cb * 32 + 16, 16)] = u1 * ev
                stg[b, e, pl.ds(64, 16)] = jnp.where(lanes == 0, ev, 0.0)
            return carry2
        lax.fori_loop(0, CHUNKB // 8, _row, 0)

    def _pair(p, carry):
        for b in range(2):
            ch = 2 * p + b
            gath_wait(b)
            idx_wait(1 - b)
            gath_start(1 - b)

            @pl.when(ch >= 2)
            def _():
                scat_wait(b)

            compute(b)
            scat_start(b)
            idx_start(ch + 2, b)
        return carry
    lax.fori_loop(0, CHB // 2, _pair, 0)

    gath_wait(0)
    idx_wait(1)
    scat_wait(0)
    scat_wait(1)

    plsc.subcore_barrier()
    pltpu.sync_copy(acc_shr.at[pl.ds(sid * 640, 640)],
                    acc_hbm.at[cid, pl.ds(sid * 640, 640)])


BLK = 1024
NBLK = NPAD // BLK


def _mm_body(x_ref, wl_ref, wr_ref, bl_ref, br_ref, xl_ref, xr_ref,
             xlb_ref, xrb_ref):
    xb = x_ref[...]
    xl = jnp.dot(xb, wl_ref[...],
                 preferred_element_type=jnp.float32) + bl_ref[...]
    xr = jnp.dot(xb, wr_ref[...],
                 preferred_element_type=jnp.float32) + br_ref[...]
    xl_ref[...] = xl
    xr_ref[...] = xr
    xlb_ref[...] = xl.astype(jnp.bfloat16)
    xrb_ref[...] = xr.astype(jnp.bfloat16)


def _mm(x_p, Wl, Wr, bl, br):
    din = x_p.shape[1]
    return pl.pallas_call(
        _mm_body,
        grid=(NBLK,),
        in_specs=[pl.BlockSpec((BLK, din), lambda i: (i, 0)),
                  pl.BlockSpec((din, 64), lambda i: (0, 0)),
                  pl.BlockSpec((din, 64), lambda i: (0, 0)),
                  pl.BlockSpec((1, 64), lambda i: (0, 0)),
                  pl.BlockSpec((1, 64), lambda i: (0, 0))],
        out_specs=[pl.BlockSpec((BLK, 64), lambda i: (i, 0)),
                   pl.BlockSpec((BLK, 64), lambda i: (i, 0)),
                   pl.BlockSpec((BLK, 64), lambda i: (i, 0)),
                   pl.BlockSpec((BLK, 64), lambda i: (i, 0))],
        out_shape=[jax.ShapeDtypeStruct((NPAD, 64), jnp.float32),
                   jax.ShapeDtypeStruct((NPAD, 64), jnp.float32),
                   jax.ShapeDtypeStruct((NPAD, 64), jnp.bfloat16),
                   jax.ShapeDtypeStruct((NPAD, 64), jnp.bfloat16)],
    )(x_p, Wl, Wr, bl.reshape(1, 64), br.reshape(1, 64))


def _fin_body(a0_ref, a1_ref, bias_ref, wl_ref, wr_ref, bl_ref, br_ref,
              xl_ref, xr_ref, xlb_ref, xrb_ref):
    a = a0_ref[...] + a1_ref[...]
    h = a[:, :64] / (a[:, 64:65] + 1e-16) + bias_ref[...]
    h = jnp.maximum(h, 0.0)
    xl = jnp.dot(h, wl_ref[...],
                 preferred_element_type=jnp.float32) + bl_ref[...]
    xr = jnp.dot(h, wr_ref[...],
                 preferred_element_type=jnp.float32) + br_ref[...]
    xl_ref[...] = xl
    xr_ref[...] = xr
    xlb_ref[...] = xl.astype(jnp.bfloat16)
    xrb_ref[...] = xr.astype(jnp.bfloat16)


def _fin(acc, bias, Wl, bl, Wr, br):
    return pl.pallas_call(
        _fin_body,
        grid=(NBLK,),
        in_specs=[pl.BlockSpec((BLK, 80), lambda i: (i, 0)),
                  pl.BlockSpec((BLK, 80), lambda i: (i, 0)),
                  pl.BlockSpec((1, 64), lambda i: (0, 0)),
                  pl.BlockSpec((64, 64), lambda i: (0, 0)),
                  pl.BlockSpec((64, 64), lambda i: (0, 0)),
                  pl.BlockSpec((1, 64), lambda i: (0, 0)),
                  pl.BlockSpec((1, 64), lambda i: (0, 0))],
        out_specs=[pl.BlockSpec((BLK, 64), lambda i: (i, 0)),
                   pl.BlockSpec((BLK, 64), lambda i: (i, 0)),
                   pl.BlockSpec((BLK, 64), lambda i: (i, 0)),
                   pl.BlockSpec((BLK, 64), lambda i: (i, 0))],
        out_shape=[jax.ShapeDtypeStruct((NPAD, 64), jnp.float32),
                   jax.ShapeDtypeStruct((NPAD, 64), jnp.float32),
                   jax.ShapeDtypeStruct((NPAD, 64), jnp.bfloat16),
                   jax.ShapeDtypeStruct((NPAD, 64), jnp.bfloat16)],
    )(acc[0], acc[1], bias.reshape(1, 64), Wl, Wr,
      bl.reshape(1, 64), br.reshape(1, 64))


def _pool_body(a0_ref, a1_ref, bias_ref, batch_ref, o_ref, sacc_ref):
    pid = pl.program_id(0)
    a = a0_ref[...] + a1_ref[...]
    h = a[:, :64] / (a[:, 64:65] + 1e-16) + bias_ref[...]
    h = jnp.where(lax.broadcasted_iota(jnp.int32, (BLK, 64), 1) == 63,
                  1.0, h)
    b = batch_ref[0]
    oh = jnp.where(b == lax.broadcasted_iota(jnp.int32, (G, BLK), 0),
                   1.0, 0.0)

    @pl.when(pid == 0)
    def _():
        sacc_ref[...] = jnp.zeros_like(sacc_ref)

    sacc_ref[...] += jnp.dot(oh, h, preferred_element_type=jnp.float32)

    @pl.when(pid == NBLK - 1)
    def _():
        s = sacc_ref[...]
        o_ref[...] = s / jnp.maximum(s[:, 63:64], 1.0)


def _pool(acc, bias, batch_p):
    return pl.pallas_call(
        _pool_body,
        grid=(NBLK,),
        in_specs=[pl.BlockSpec((BLK, 80), lambda i: (i, 0)),
                  pl.BlockSpec((BLK, 80), lambda i: (i, 0)),
                  pl.BlockSpec((1, 64), lambda i: (0, 0)),
                  pl.BlockSpec((1, 1, BLK), lambda i: (i, 0, 0))],
        out_specs=pl.BlockSpec((G, 64), lambda i: (0, 0)),
        out_shape=jax.ShapeDtypeStruct((G, 64), jnp.float32),
        scratch_shapes=[pltpu.VMEM((G, 64), jnp.float32)],
    )(acc[0], acc[1], bias.reshape(1, 64),
      batch_p.reshape(NBLK, 1, BLK))


def _pad_rows(a, rows):
    return jnp.pad(a, ((0, rows - a.shape[0]), (0, 0)))


def kernel(x, edge_index, batch, W_l0, b_l0, W_r0, b_r0, att0, bias0,
           W_l1, b_l1, W_r1, b_r1, att1, bias1,
           W_l2, b_l2, W_r2, b_r2, att2, bias2):
    loops = jnp.arange(N, dtype=jnp.int32)
    src = jnp.concatenate([edge_index[0].astype(jnp.int32), loops,
                           jnp.zeros((E_PAD - ET,), jnp.int32)])
    dst = jnp.concatenate([edge_index[1].astype(jnp.int32), loops,
                           jnp.full((E_PAD - ET,), N, jnp.int32)])
    batch_p = jnp.concatenate([batch.astype(jnp.int32),
                               jnp.full((NPAD - N,), 300, jnp.int32)])
    x_p = _pad_rows(x, NPAD)

    pad6 = lambda a: jnp.pad(a, ((0, 0), (0, 6)))
    Wl2, Wr2 = pad6(W_l2), pad6(W_r2)
    bl2 = jnp.pad(b_l2, (0, 6))
    br2 = jnp.pad(b_r2, (0, 6))
    att2p = jnp.pad(att2[0], (0, 6))
    bias2p = jnp.pad(bias2, (0, 6))

    def att_perm(a):
        a2 = a.reshape(2, 16, 2)
        return jnp.concatenate([a2[:, :, 0], a2[:, :, 1]],
                               axis=1).reshape(64)

    perm = jnp.concatenate([jnp.arange(0, 32, 2), jnp.arange(1, 32, 2),
                            jnp.arange(32, 64, 2), jnp.arange(33, 64, 2)])
    inv = jnp.argsort(perm)

    xl, xr, xlb, xrb = _mm(x_p, W_l0, W_r0, b_l0, b_r0)
    alpha, amax_parts = _edge_alpha(xlb, xrb, src, dst, att_perm(att0[0]))
    acc = _edge_scatter(xlb, src, dst, alpha, amax_parts)
    xl, xr, xlb, xrb = _fin(acc, bias0[perm], W_l1[perm], b_l1,
                            W_r1[perm], b_r1)
    alpha, amax_parts = _edge_alpha(xlb, xrb, src, dst, att_perm(att1[0]))
    acc = _edge_scatter(xlb, src, dst, alpha, amax_parts)
    xl, xr, xlb, xrb = _fin(acc, bias1[perm], Wl2[perm], bl2,
                            Wr2[perm], br2)
    alpha, amax_parts = _edge_alpha(xlb, xrb, src, dst, att_perm(att2p))
    acc = _edge_scatter(xlb, src, dst, alpha, amax_parts)
    out = _pool(acc, bias2p[perm], batch_p)
    return out[:, inv][:, :58]
